# baseline (speedup 1.0000x reference)
import functools

import jax
import jax.numpy as jnp
from jax import lax
from jax.experimental import pallas as pl
from jax.experimental.pallas import tpu as pltpu
from jax.experimental.pallas import tpu_sc as plsc

_N = 4096 * 20
_D = 1000
_NC, _NS = 2, 16
_NW = _NC * _NS
_PER_W = _N // _NW
_K = 40
_CHUNKS = _PER_W // _K
_NBUF = 3


def _sc_gather(x_flat, table):
    mesh = plsc.VectorSubcoreMesh(core_axis_name="c", subcore_axis_name="s")

    @functools.partial(
        pl.kernel,
        mesh=mesh,
        out_type=jax.ShapeDtypeStruct((_N, _D), jnp.float32),
        compiler_params=pltpu.CompilerParams(use_tc_tiling_on_sc=False),
        scratch_types=[
            pltpu.VMEM((_PER_W,), jnp.int32),
            pltpu.VMEM((_NBUF, _K, _D), jnp.float32),
            pltpu.SemaphoreType.DMA,
            pltpu.SemaphoreType.DMA,
        ],
    )
    def k(idx_hbm, table_hbm, out_hbm, idx_v, rows_v, gsem, wsem):
        wid = lax.axis_index("s") * _NC + lax.axis_index("c")
        base = wid * _PER_W

        pltpu.sync_copy(idx_hbm.at[pl.ds(base, _PER_W)], idx_v)

        def fire(g, slot):
            pltpu.async_copy(table_hbm.at[idx_v.at[pl.ds(g * _K, _K)]],
                             rows_v.at[slot], gsem)

        def wait_gather(g, slot):
            pltpu.make_async_copy(table_hbm.at[idx_v.at[pl.ds(g * _K, _K)]],
                                  rows_v.at[slot], gsem).wait()

        def issue_write(g, slot):
            pltpu.async_copy(rows_v.at[slot],
                             out_hbm.at[pl.ds(base + g * _K, _K)], wsem)

        def wait_write(g, slot):
            pltpu.make_async_copy(rows_v.at[slot],
                                  out_hbm.at[pl.ds(base + g * _K, _K)],
                                  wsem).wait()

        for c in range(_NBUF - 1):
            fire(c, c)

        def body(g, _):
            slot = lax.rem(g, _NBUF)

            @pl.when(g >= 1)
            def _():
                wait_write(g - 1, lax.rem(g - 1, _NBUF))

            @pl.when(g + _NBUF - 1 < _CHUNKS)
            def _():
                fire(g + _NBUF - 1, lax.rem(g + _NBUF - 1, _NBUF))

            wait_gather(g, slot)
            issue_write(g, slot)
            return 0

        lax.fori_loop(0, _CHUNKS, body, 0)

        wait_write(_CHUNKS - 1, lax.rem(_CHUNKS - 1, _NBUF))

    return k(x_flat, table)


def kernel(x, table):
    xf = x.reshape(-1).astype(jnp.int32)
    out = _sc_gather(xf, table)
    return out.reshape(x.shape + (table.shape[0],))


# speedup vs baseline: 1.1478x; 1.1478x over previous
import functools

import jax
import jax.numpy as jnp
from jax import lax
from jax.experimental import pallas as pl
from jax.experimental.pallas import tpu as pltpu
from jax.experimental.pallas import tpu_sc as plsc

_N = 4096 * 20
_D = 1000
_NC, _NS = 2, 16
_NW = _NC * _NS
_PER_W = _N // _NW
_K = 32
_CHUNKS = _PER_W // _K
_NBUF = 2


def _sc_gather(x_flat, table):
    mesh = plsc.VectorSubcoreMesh(core_axis_name="c", subcore_axis_name="s")

    @functools.partial(
        pl.kernel,
        mesh=mesh,
        out_type=jax.ShapeDtypeStruct((_N, _D), jnp.float32),
        compiler_params=pltpu.CompilerParams(use_tc_tiling_on_sc=False),
        scratch_types=[
            pltpu.VMEM((_PER_W,), jnp.int32),
            pltpu.VMEM((_NBUF, _K, _D), jnp.float32),
            pltpu.VMEM_SHARED((1000, _D), jnp.float32),
            pltpu.SemaphoreType.DMA,
            pltpu.SemaphoreType.DMA,
        ],
    )
    def k(idx_hbm, table_hbm, out_hbm, idx_v, rows_v, table_sp, gsem, wsem):
        wid = lax.axis_index("s") * _NC + lax.axis_index("c")
        base = wid * _PER_W

        @pl.when(lax.axis_index("s") == 0)
        def _():
            pltpu.sync_copy(table_hbm, table_sp)

        pltpu.sync_copy(idx_hbm.at[pl.ds(base, _PER_W)], idx_v)
        plsc.subcore_barrier()

        def fire(g, slot):
            pltpu.async_copy(table_sp.at[idx_v.at[pl.ds(g * _K, _K)]],
                             rows_v.at[slot], gsem)

        def wait_gather(g, slot):
            pltpu.make_async_copy(table_sp.at[idx_v.at[pl.ds(g * _K, _K)]],
                                  rows_v.at[slot], gsem).wait()

        def issue_write(g, slot):
            pltpu.async_copy(rows_v.at[slot],
                             out_hbm.at[pl.ds(base + g * _K, _K)], wsem)

        def wait_write(g, slot):
            pltpu.make_async_copy(rows_v.at[slot],
                                  out_hbm.at[pl.ds(base + g * _K, _K)],
                                  wsem).wait()

        for c in range(_NBUF - 1):
            fire(c, c)

        def body(g, _):
            slot = lax.rem(g, _NBUF)

            @pl.when(g >= 1)
            def _():
                wait_write(g - 1, lax.rem(g - 1, _NBUF))

            @pl.when(g + _NBUF - 1 < _CHUNKS)
            def _():
                fire(g + _NBUF - 1, lax.rem(g + _NBUF - 1, _NBUF))

            wait_gather(g, slot)
            issue_write(g, slot)
            return 0

        lax.fori_loop(0, _CHUNKS, body, 0)

        wait_write(_CHUNKS - 1, lax.rem(_CHUNKS - 1, _NBUF))

    return k(x_flat, table)


def kernel(x, table):
    xf = x.reshape(-1).astype(jnp.int32)
    out = _sc_gather(xf, table)
    return out.reshape(x.shape + (table.shape[0],))
